# baseline (speedup 1.0000x reference)
import functools

import jax
import jax.numpy as jnp
from jax import lax
from jax.experimental import pallas as pl
from jax.experimental.pallas import tpu as pltpu
from jax.experimental.pallas import tpu_sc as plsc

NC, NS, NLANE = 2, 16, 16
CH = 128
ZR = 128


def _fill_const(ref, rows, cols, val):
    groups = cols // NLANE

    def body(i, carry):
        r = i // groups
        g = i % groups
        ref[r, pl.ds(g * NLANE, NLANE)] = jnp.full((NLANE,), val, jnp.float32)
        return carry

    lax.fori_loop(0, rows * groups, body, 0)


def _pad_rows(n):
    return ZR * ((n + NS * ZR - 1) // (NS * ZR))


def _make_sc_agg(n, ep, d):
    n_chunks = ep // CH
    per_w = n_chunks // (NC * NS)
    rows_per_tile = _pad_rows(n)
    n_pad = NS * rows_per_tile
    nzc = rows_per_tile // ZR
    assert n_chunks % (NC * NS) == 0 and per_w % 2 == 0 and n_pad > n

    mesh = plsc.VectorSubcoreMesh(core_axis_name="c", subcore_axis_name="s")

    @functools.partial(
        pl.kernel,
        out_type=jax.ShapeDtypeStruct((NC, n_pad, d), jnp.float32),
        mesh=mesh,
        scratch_types=[
            pltpu.VMEM((2, 2, CH), jnp.int32),
            pltpu.VMEM((CH, d), jnp.float32),
            pltpu.VMEM((CH, d), jnp.float32),
            pltpu.VMEM_SHARED((n_pad, d), jnp.float32),
            pltpu.SemaphoreType.DMA,
            pltpu.SemaphoreType.DMA,
        ],
    )
    def agg(u_hbm, idx_hbm, out_hbm, idx_v, msg0, msg1, acc, gsem, ssem):
        cid = lax.axis_index("c")
        sid = lax.axis_index("s")
        wid = cid * NS + sid
        msgs = (msg0, msg1)
        _fill_const(msg0, ZR, d, 0.0)
        base_r = sid * rows_per_tile
        for j in range(nzc):
            pltpu.sync_copy(msg0, acc.at[pl.ds(base_r + j * ZR, ZR)])
        plsc.subcore_barrier()

        c0 = wid * per_w
        pltpu.sync_copy(idx_hbm.at[c0], idx_v.at[0])
        pltpu.async_copy(u_hbm.at[idx_v.at[0, 0]], msg0, gsem)

        def body(t, carry):
            for p in range(2):
                i = 2 * t + p
                q = 1 - p

                @pl.when(i > 0)
                def _():
                    pltpu.make_async_copy(
                        msgs[q], acc.at[idx_v.at[q, 1]], ssem).wait()

                @pl.when(i + 1 < per_w)
                def _():
                    pltpu.sync_copy(idx_hbm.at[c0 + i + 1], idx_v.at[q])
                    pltpu.async_copy(u_hbm.at[idx_v.at[q, 0]], msgs[q], gsem)

                pltpu.make_async_copy(
                    u_hbm.at[idx_v.at[p, 0]], msgs[p], gsem).wait()
                pltpu.async_copy(msgs[p], acc.at[idx_v.at[p, 1]], ssem, add=True)
            return carry

        lax.fori_loop(0, per_w // 2, body, 0)
        pltpu.make_async_copy(msgs[1], acc.at[idx_v.at[1, 1]], ssem).wait()
        plsc.subcore_barrier()
        pltpu.sync_copy(acc.at[pl.ds(base_r, rows_per_tile)],
                        out_hbm.at[cid, pl.ds(base_r, rows_per_tile)])

    return agg


def _make_sc_deg(n, ep):
    d = NLANE
    n_chunks = ep // CH
    per_w = n_chunks // (NC * NS)
    rows_per_tile = _pad_rows(n)
    n_pad = NS * rows_per_tile
    nzc = rows_per_tile // ZR

    mesh = plsc.VectorSubcoreMesh(core_axis_name="c", subcore_axis_name="s")

    @functools.partial(
        pl.kernel,
        out_type=jax.ShapeDtypeStruct((NC, n_pad, d), jnp.float32),
        mesh=mesh,
        compiler_params=pltpu.CompilerParams(use_tc_tiling_on_sc=False),
        scratch_types=[
            pltpu.VMEM((2, 2, CH), jnp.int32),
            pltpu.VMEM((CH, d), jnp.float32),
            pltpu.VMEM((ZR, d), jnp.float32),
            pltpu.VMEM_SHARED((n_pad, d), jnp.float32),
            pltpu.SemaphoreType.DMA,
        ],
    )
    def deg(idx_hbm, out_hbm, idx_v, ones_v, zero_v, acc, ssem):
        cid = lax.axis_index("c")
        sid = lax.axis_index("s")
        wid = cid * NS + sid
        _fill_const(zero_v, ZR, d, 0.0)
        _fill_const(ones_v, CH, d, 1.0)
        base_r = sid * rows_per_tile
        for j in range(nzc):
            pltpu.sync_copy(zero_v, acc.at[pl.ds(base_r + j * ZR, ZR)])
        plsc.subcore_barrier()

        c0 = wid * per_w
        pltpu.sync_copy(idx_hbm.at[c0], idx_v.at[0])

        def body(t, carry):
            for p in range(2):
                i = 2 * t + p
                q = 1 - p

                @pl.when(i > 0)
                def _():
                    pltpu.make_async_copy(
                        ones_v, acc.at[idx_v.at[q, 1]], ssem).wait()

                pltpu.async_copy(ones_v, acc.at[idx_v.at[p, 1]], ssem, add=True)

                @pl.when(i + 1 < per_w)
                def _():
                    pltpu.sync_copy(idx_hbm.at[c0 + i + 1], idx_v.at[q])
            return carry

        lax.fori_loop(0, per_w // 2, body, 0)
        pltpu.make_async_copy(ones_v, acc.at[idx_v.at[1, 1]], ssem).wait()
        plsc.subcore_barrier()
        pltpu.sync_copy(acc.at[pl.ds(base_r, rows_per_tile)],
                        out_hbm.at[cid, pl.ds(base_r, rows_per_tile)])

    return deg


def _elu(x):
    return jnp.where(x > 0, x, jnp.exp(x) - 1.0)


def _prep_body(dg0, dg1, x, q_out, dinv_out):
    deg = dg0[0][:, 0:1] + dg1[0][:, 0:1] + 1.0
    dinv = lax.rsqrt(deg)
    q_out[...] = x[...] * dinv
    dinv_out[...] = jnp.broadcast_to(dinv, dinv_out.shape)


def _l1_body(a0, a1, q, dinv, W1, b1, fc1W, fc1b, h2_out, mom_out):
    t = a0[0] + a1[0] + q[...]
    h = _elu((jnp.dot(t, W1[...], preferred_element_type=jnp.float32)
              * dinv[:, 0:1]) + b1[...])
    h2 = _elu(jnp.dot(h, fc1W[...], preferred_element_type=jnp.float32) + fc1b[...])
    h2_out[...] = h2
    s1 = jnp.sum(h2, axis=0, keepdims=True)
    s2 = jnp.sum(h2 * h2, axis=0, keepdims=True)
    delta = jnp.concatenate([s1, s2], axis=0)

    @pl.when(pl.program_id(0) == 0)
    def _():
        mom_out[...] = jnp.zeros_like(mom_out)

    mom_out[...] += delta


def _make_l2_body(n):
    def _l2_body(h2, mom, dinv, W2, u2_out):
        m = mom[...]
        mean = m[0:1, :] * (1.0 / n)
        var = m[1:2, :] * (1.0 / n) - mean * mean
        s = lax.rsqrt(var + 1e-5)
        y = (h2[...] - mean) * s
        u2_out[...] = (jnp.dot(y, W2[...], preferred_element_type=jnp.float32)
                       * dinv[:, 0:1])
    return _l2_body


def _l2post_body(a0, a1, u2, dinv, b2, fc2W, fc2b, p3_out):
    dv = dinv[:, 0:1]
    y2 = _elu(dv * (a0[0] + a1[0] + u2[...]) + b2[...])
    y3 = _elu(jnp.dot(y2, fc2W[...], preferred_element_type=jnp.float32) + fc2b[...])
    p3_out[...] = y3 * dv


def _out_body(a0, a1, p3, dinv, W3p, b3p, fc3Wp, fc3bp, z_out):
    dv = dinv[:, 0:1]
    t3 = a0[0] + a1[0] + p3[...]
    z = _elu(dv * jnp.dot(t3, W3p[...], preferred_element_type=jnp.float32) + b3p[...])
    z_out[...] = jnp.dot(z, fc3Wp[...], preferred_element_type=jnp.float32) + fc3bp[...]


def _rows_spec(rb, dcol, offset_blocks=0):
    return pl.BlockSpec((rb, dcol), lambda r: (r + offset_blocks, 0))


def _part_spec(rb, dcol, c):
    return pl.BlockSpec((1, rb, dcol), lambda r: (c, r, 0))


def _full_spec(shape):
    return pl.BlockSpec(shape, lambda r: tuple(0 for _ in shape))


def kernel(x, adj, num_graphs, in_batch, cluster,
           W1, b1, fc1W, fc1b, W2, b2, fc2W, fc2b, W3, b3, fc3W, fc3b):
    n, f = x.shape
    e = adj.shape[1]
    d1 = W1.shape[1]
    d2 = W2.shape[1]
    d3p = NLANE

    epw = NC * NS * CH * 2
    ep = ((e + epw - 1) // epw) * epw
    row = adj[0]
    col = adj[1]
    if ep != e:
        spare = NS * _pad_rows(n) - n
        dummy = n + jnp.arange(ep - e, dtype=col.dtype) % min(CH, spare)
        row = jnp.concatenate([row, jnp.zeros((ep - e,), row.dtype)])
        col = jnp.concatenate([col, dummy])
    packed = jnp.stack([row.reshape(-1, CH), col.reshape(-1, CH)], axis=1)

    rb = 1000
    grid = (n // rb,)

    degp = _make_sc_deg(n, ep)(packed)

    q, dinv = pl.pallas_call(
        _prep_body,
        grid=grid,
        in_specs=[_part_spec(rb, d3p, 0), _part_spec(rb, d3p, 1),
                  _rows_spec(rb, f)],
        out_specs=[_rows_spec(rb, f), _rows_spec(rb, 8)],
        out_shape=[jax.ShapeDtypeStruct((n, f), jnp.float32),
                   jax.ShapeDtypeStruct((n, 8), jnp.float32)],
    )(degp, degp, x)

    agg128 = _make_sc_agg(n, ep, f)
    a1p = agg128(q, packed)

    h2, mom = pl.pallas_call(
        _l1_body,
        grid=grid,
        in_specs=[_part_spec(rb, f, 0), _part_spec(rb, f, 1),
                  _rows_spec(rb, f), _rows_spec(rb, 8),
                  _full_spec(W1.shape), _full_spec((1, d1)),
                  _full_spec(fc1W.shape), _full_spec((1, d1))],
        out_specs=[_rows_spec(rb, d1), _full_spec((2, d1))],
        out_shape=[jax.ShapeDtypeStruct((n, d1), jnp.float32),
                   jax.ShapeDtypeStruct((2, d1), jnp.float32)],
    )(a1p, a1p, q, dinv, W1, b1.reshape(1, -1), fc1W, fc1b.reshape(1, -1))

    u2 = pl.pallas_call(
        _make_l2_body(float(n)),
        grid=grid,
        in_specs=[_rows_spec(rb, d1), _full_spec((2, d1)),
                  _rows_spec(rb, 8), _full_spec(W2.shape)],
        out_specs=_rows_spec(rb, d2),
        out_shape=jax.ShapeDtypeStruct((n, d2), jnp.float32),
    )(h2, mom, dinv, W2)

    a2p = agg128(u2, packed)

    p3 = pl.pallas_call(
        _l2post_body,
        grid=grid,
        in_specs=[_part_spec(rb, d2, 0), _part_spec(rb, d2, 1),
                  _rows_spec(rb, d2), _rows_spec(rb, 8),
                  _full_spec((1, d2)), _full_spec(fc2W.shape),
                  _full_spec((1, d2))],
        out_specs=_rows_spec(rb, d2),
        out_shape=jax.ShapeDtypeStruct((n, d2), jnp.float32),
    )(a2p, a2p, u2, dinv, b2.reshape(1, -1), fc2W, fc2b.reshape(1, -1))

    a3p = agg128(p3, packed)

    nout = W3.shape[1]
    W3p = jnp.zeros((W3.shape[0], d3p), jnp.float32).at[:, :nout].set(W3)
    b3p = jnp.zeros((1, d3p), jnp.float32).at[0, :nout].set(b3)
    fc3Wp = jnp.zeros((d3p, d3p), jnp.float32).at[:nout, :nout].set(fc3W)
    fc3bp = jnp.zeros((1, d3p), jnp.float32).at[0, :nout].set(fc3b)
    zp = pl.pallas_call(
        _out_body,
        grid=grid,
        in_specs=[_part_spec(rb, d2, 0), _part_spec(rb, d2, 1),
                  _rows_spec(rb, d2), _rows_spec(rb, 8),
                  _full_spec((W3.shape[0], d3p)), _full_spec((1, d3p)),
                  _full_spec((d3p, d3p)), _full_spec((1, d3p))],
        out_specs=_rows_spec(rb, d3p),
        out_shape=jax.ShapeDtypeStruct((n, d3p), jnp.float32),
    )(a3p, a3p, p3, dinv, W3p, b3p, fc3Wp, fc3bp)

    return zp[:, :nout]


# speedup vs baseline: 1.0471x; 1.0471x over previous
import functools

import jax
import jax.numpy as jnp
from jax import lax
from jax.experimental import pallas as pl
from jax.experimental.pallas import tpu as pltpu
from jax.experimental.pallas import tpu_sc as plsc

NC, NS, NLANE = 2, 16, 16
CH = 128
ZR = 128
_SC0_FRAC = 0.76


def _fill_const(ref, rows, cols, val):
    groups = cols // NLANE

    def body(i, carry):
        r = i // groups
        g = i % groups
        ref[r, pl.ds(g * NLANE, NLANE)] = jnp.full((NLANE,), val, jnp.float32)
        return carry

    lax.fori_loop(0, rows * groups, body, 0)


def _pad_rows(n):
    return ZR * ((n + NS * ZR - 1) // (NS * ZR))


def _make_sc_agg(n, ep, d):
    n_chunks = ep // CH
    per_sc = n_chunks // NC
    w0 = 2 * int(round(_SC0_FRAC * (n_chunks // NS) / 2.0))
    w1 = n_chunks // NS - w0
    rows_per_tile = _pad_rows(n)
    n_pad = NS * rows_per_tile
    nzc = rows_per_tile // ZR
    assert n_chunks % (NC * NS) == 0 and w0 % 2 == 0 and w1 % 2 == 0
    assert w0 >= 2 and w1 >= 2 and n_pad > n

    mesh = plsc.VectorSubcoreMesh(core_axis_name="c", subcore_axis_name="s")

    @functools.partial(
        pl.kernel,
        out_type=jax.ShapeDtypeStruct((NC, n_pad, d), jnp.float32),
        mesh=mesh,
        scratch_types=[
            pltpu.VMEM((2, 2, CH), jnp.int32),
            pltpu.VMEM((CH, d), jnp.float32),
            pltpu.VMEM((CH, d), jnp.float32),
            pltpu.VMEM_SHARED((n_pad, d), jnp.float32),
            pltpu.SemaphoreType.DMA,
            pltpu.SemaphoreType.DMA,
        ],
    )
    def agg(u_hbm, idx_hbm, out_hbm, idx_v, msg0, msg1, acc, gsem, ssem):
        cid = lax.axis_index("c")
        sid = lax.axis_index("s")
        msgs = (msg0, msg1)
        _fill_const(msg0, ZR, d, 0.0)
        base_r = sid * rows_per_tile
        for j in range(nzc):
            pltpu.sync_copy(msg0, acc.at[pl.ds(base_r + j * ZR, ZR)])
        plsc.subcore_barrier()

        my_w = jnp.where(cid == 0, w0, w1)
        c0 = cid * NS * w0 + sid * my_w
        pltpu.sync_copy(idx_hbm.at[c0], idx_v.at[0])
        pltpu.async_copy(u_hbm.at[idx_v.at[0, 0]], msg0, gsem)

        def body(t, carry):
            for p in range(2):
                i = 2 * t + p
                q = 1 - p

                @pl.when(i > 0)
                def _():
                    pltpu.make_async_copy(
                        msgs[q], acc.at[idx_v.at[q, 1]], ssem).wait()

                @pl.when(i + 1 < my_w)
                def _():
                    pltpu.sync_copy(idx_hbm.at[c0 + i + 1], idx_v.at[q])
                    pltpu.async_copy(u_hbm.at[idx_v.at[q, 0]], msgs[q], gsem)

                pltpu.make_async_copy(
                    u_hbm.at[idx_v.at[p, 0]], msgs[p], gsem).wait()
                pltpu.async_copy(msgs[p], acc.at[idx_v.at[p, 1]], ssem, add=True)
            return carry

        lax.fori_loop(0, my_w // 2, body, 0)
        pltpu.make_async_copy(msgs[1], acc.at[idx_v.at[1, 1]], ssem).wait()
        plsc.subcore_barrier()
        pltpu.sync_copy(acc.at[pl.ds(base_r, rows_per_tile)],
                        out_hbm.at[cid, pl.ds(base_r, rows_per_tile)])

    return agg


def _make_sc_deg(n, ep):
    d = NLANE
    n_chunks = ep // CH
    per_w = n_chunks // (NC * NS)
    rows_per_tile = _pad_rows(n)
    n_pad = NS * rows_per_tile
    nzc = rows_per_tile // ZR

    mesh = plsc.VectorSubcoreMesh(core_axis_name="c", subcore_axis_name="s")

    @functools.partial(
        pl.kernel,
        out_type=jax.ShapeDtypeStruct((NC, n_pad, d), jnp.float32),
        mesh=mesh,
        compiler_params=pltpu.CompilerParams(use_tc_tiling_on_sc=False),
        scratch_types=[
            pltpu.VMEM((2, 2, CH), jnp.int32),
            pltpu.VMEM((CH, d), jnp.float32),
            pltpu.VMEM((ZR, d), jnp.float32),
            pltpu.VMEM_SHARED((n_pad, d), jnp.float32),
            pltpu.SemaphoreType.DMA,
        ],
    )
    def deg(idx_hbm, out_hbm, idx_v, ones_v, zero_v, acc, ssem):
        cid = lax.axis_index("c")
        sid = lax.axis_index("s")
        wid = cid * NS + sid
        _fill_const(zero_v, ZR, d, 0.0)
        _fill_const(ones_v, CH, d, 1.0)
        base_r = sid * rows_per_tile
        for j in range(nzc):
            pltpu.sync_copy(zero_v, acc.at[pl.ds(base_r + j * ZR, ZR)])
        plsc.subcore_barrier()

        c0 = wid * per_w
        pltpu.sync_copy(idx_hbm.at[c0], idx_v.at[0])

        def body(t, carry):
            for p in range(2):
                i = 2 * t + p
                q = 1 - p

                @pl.when(i > 0)
                def _():
                    pltpu.make_async_copy(
                        ones_v, acc.at[idx_v.at[q, 1]], ssem).wait()

                pltpu.async_copy(ones_v, acc.at[idx_v.at[p, 1]], ssem, add=True)

                @pl.when(i + 1 < per_w)
                def _():
                    pltpu.sync_copy(idx_hbm.at[c0 + i + 1], idx_v.at[q])
            return carry

        lax.fori_loop(0, per_w // 2, body, 0)
        pltpu.make_async_copy(ones_v, acc.at[idx_v.at[1, 1]], ssem).wait()
        plsc.subcore_barrier()
        pltpu.sync_copy(acc.at[pl.ds(base_r, rows_per_tile)],
                        out_hbm.at[cid, pl.ds(base_r, rows_per_tile)])

    return deg


def _elu(x):
    return jnp.where(x > 0, x, jnp.exp(x) - 1.0)


def _prep_body(dg0, dg1, x, q_out, dinv_out):
    deg = dg0[0][:, 0:1] + dg1[0][:, 0:1] + 1.0
    dinv = lax.rsqrt(deg)
    q_out[...] = x[...] * dinv
    dinv_out[...] = jnp.broadcast_to(dinv, dinv_out.shape)


def _l1_body(a0, a1, q, dinv, W1, b1, fc1W, fc1b, h2_out, mom_out):
    t = a0[0] + a1[0] + q[...]
    h = _elu((jnp.dot(t, W1[...], preferred_element_type=jnp.float32)
              * dinv[:, 0:1]) + b1[...])
    h2 = _elu(jnp.dot(h, fc1W[...], preferred_element_type=jnp.float32) + fc1b[...])
    h2_out[...] = h2
    s1 = jnp.sum(h2, axis=0, keepdims=True)
    s2 = jnp.sum(h2 * h2, axis=0, keepdims=True)
    delta = jnp.concatenate([s1, s2], axis=0)

    @pl.when(pl.program_id(0) == 0)
    def _():
        mom_out[...] = jnp.zeros_like(mom_out)

    mom_out[...] += delta


def _make_l2_body(n):
    def _l2_body(h2, mom, dinv, W2, u2_out):
        m = mom[...]
        mean = m[0:1, :] * (1.0 / n)
        var = m[1:2, :] * (1.0 / n) - mean * mean
        s = lax.rsqrt(var + 1e-5)
        y = (h2[...] - mean) * s
        u2_out[...] = (jnp.dot(y, W2[...], preferred_element_type=jnp.float32)
                       * dinv[:, 0:1])
    return _l2_body


def _l2post_body(a0, a1, u2, dinv, b2, fc2W, fc2b, p3_out):
    dv = dinv[:, 0:1]
    y2 = _elu(dv * (a0[0] + a1[0] + u2[...]) + b2[...])
    y3 = _elu(jnp.dot(y2, fc2W[...], preferred_element_type=jnp.float32) + fc2b[...])
    p3_out[...] = y3 * dv


def _out_body(a0, a1, p3, dinv, W3p, b3p, fc3Wp, fc3bp, z_out):
    dv = dinv[:, 0:1]
    t3 = a0[0] + a1[0] + p3[...]
    z = _elu(dv * jnp.dot(t3, W3p[...], preferred_element_type=jnp.float32) + b3p[...])
    z_out[...] = jnp.dot(z, fc3Wp[...], preferred_element_type=jnp.float32) + fc3bp[...]


def _rows_spec(rb, dcol, offset_blocks=0):
    return pl.BlockSpec((rb, dcol), lambda r: (r + offset_blocks, 0))


def _part_spec(rb, dcol, c):
    return pl.BlockSpec((1, rb, dcol), lambda r: (c, r, 0))


def _full_spec(shape):
    return pl.BlockSpec(shape, lambda r: tuple(0 for _ in shape))


def kernel(x, adj, num_graphs, in_batch, cluster,
           W1, b1, fc1W, fc1b, W2, b2, fc2W, fc2b, W3, b3, fc3W, fc3b):
    n, f = x.shape
    e = adj.shape[1]
    d1 = W1.shape[1]
    d2 = W2.shape[1]
    d3p = NLANE

    epw = NC * NS * CH * 2
    ep = ((e + epw - 1) // epw) * epw
    row = adj[0]
    col = adj[1]
    if ep != e:
        spare = NS * _pad_rows(n) - n
        dummy = n + jnp.arange(ep - e, dtype=col.dtype) % min(CH, spare)
        row = jnp.concatenate([row, jnp.zeros((ep - e,), row.dtype)])
        col = jnp.concatenate([col, dummy])
    packed = jnp.stack([row.reshape(-1, CH), col.reshape(-1, CH)], axis=1)

    rb = 1000
    grid = (n // rb,)

    degp = _make_sc_deg(n, ep)(packed)

    q, dinv = pl.pallas_call(
        _prep_body,
        grid=grid,
        in_specs=[_part_spec(rb, d3p, 0), _part_spec(rb, d3p, 1),
                  _rows_spec(rb, f)],
        out_specs=[_rows_spec(rb, f), _rows_spec(rb, 8)],
        out_shape=[jax.ShapeDtypeStruct((n, f), jnp.float32),
                   jax.ShapeDtypeStruct((n, 8), jnp.float32)],
    )(degp, degp, x)

    agg128 = _make_sc_agg(n, ep, f)
    a1p = agg128(q, packed)

    h2, mom = pl.pallas_call(
        _l1_body,
        grid=grid,
        in_specs=[_part_spec(rb, f, 0), _part_spec(rb, f, 1),
                  _rows_spec(rb, f), _rows_spec(rb, 8),
                  _full_spec(W1.shape), _full_spec((1, d1)),
                  _full_spec(fc1W.shape), _full_spec((1, d1))],
        out_specs=[_rows_spec(rb, d1), _full_spec((2, d1))],
        out_shape=[jax.ShapeDtypeStruct((n, d1), jnp.float32),
                   jax.ShapeDtypeStruct((2, d1), jnp.float32)],
    )(a1p, a1p, q, dinv, W1, b1.reshape(1, -1), fc1W, fc1b.reshape(1, -1))

    u2 = pl.pallas_call(
        _make_l2_body(float(n)),
        grid=grid,
        in_specs=[_rows_spec(rb, d1), _full_spec((2, d1)),
                  _rows_spec(rb, 8), _full_spec(W2.shape)],
        out_specs=_rows_spec(rb, d2),
        out_shape=jax.ShapeDtypeStruct((n, d2), jnp.float32),
    )(h2, mom, dinv, W2)

    a2p = agg128(u2, packed)

    p3 = pl.pallas_call(
        _l2post_body,
        grid=grid,
        in_specs=[_part_spec(rb, d2, 0), _part_spec(rb, d2, 1),
                  _rows_spec(rb, d2), _rows_spec(rb, 8),
                  _full_spec((1, d2)), _full_spec(fc2W.shape),
                  _full_spec((1, d2))],
        out_specs=_rows_spec(rb, d2),
        out_shape=jax.ShapeDtypeStruct((n, d2), jnp.float32),
    )(a2p, a2p, u2, dinv, b2.reshape(1, -1), fc2W, fc2b.reshape(1, -1))

    a3p = agg128(p3, packed)

    nout = W3.shape[1]
    W3p = jnp.zeros((W3.shape[0], d3p), jnp.float32).at[:, :nout].set(W3)
    b3p = jnp.zeros((1, d3p), jnp.float32).at[0, :nout].set(b3)
    fc3Wp = jnp.zeros((d3p, d3p), jnp.float32).at[:nout, :nout].set(fc3W)
    fc3bp = jnp.zeros((1, d3p), jnp.float32).at[0, :nout].set(fc3b)
    zp = pl.pallas_call(
        _out_body,
        grid=grid,
        in_specs=[_part_spec(rb, d2, 0), _part_spec(rb, d2, 1),
                  _rows_spec(rb, d2), _rows_spec(rb, 8),
                  _full_spec((W3.shape[0], d3p)), _full_spec((1, d3p)),
                  _full_spec((d3p, d3p)), _full_spec((1, d3p))],
        out_specs=_rows_spec(rb, d3p),
        out_shape=jax.ShapeDtypeStruct((n, d3p), jnp.float32),
    )(a3p, a3p, p3, dinv, W3p, b3p, fc3Wp, fc3bp)

    return zp[:, :nout]


# speedup vs baseline: 2.1183x; 2.0230x over previous
import functools

import jax
import jax.numpy as jnp
from jax import lax
from jax.experimental import pallas as pl
from jax.experimental.pallas import tpu as pltpu
from jax.experimental.pallas import tpu_sc as plsc

NC, NS, NLANE = 2, 16, 16
CH = 128
ZR = 128
_SC0_FRAC = 0.76


def _fill_const(ref, rows, cols, val):
    groups = cols // NLANE

    def body(i, carry):
        r = i // groups
        g = i % groups
        ref[r, pl.ds(g * NLANE, NLANE)] = jnp.full((NLANE,), val, jnp.float32)
        return carry

    lax.fori_loop(0, rows * groups, body, 0)


def _pad_rows(n):
    return ZR * ((n + NS * ZR - 1) // (NS * ZR))


def _make_sc_agg(n, ep, d):
    n_chunks = ep // CH
    per_sc = n_chunks // NC
    w0 = 2 * int(round(_SC0_FRAC * (n_chunks // NS) / 2.0))
    w1 = n_chunks // NS - w0
    rows_per_tile = _pad_rows(n)
    n_pad = NS * rows_per_tile
    nzc = rows_per_tile // ZR
    assert n_chunks % (NC * NS) == 0 and w0 % 2 == 0 and w1 % 2 == 0
    assert w0 >= 2 and w1 >= 2 and n_pad > n

    mesh = plsc.VectorSubcoreMesh(core_axis_name="c", subcore_axis_name="s")

    @functools.partial(
        pl.kernel,
        out_type=jax.ShapeDtypeStruct((NC, n_pad, d), jnp.float32),
        mesh=mesh,
        scratch_types=[
            pltpu.VMEM((2, 2, CH), jnp.int32),
            pltpu.VMEM((CH, d), jnp.float32),
            pltpu.VMEM((CH, d), jnp.float32),
            pltpu.VMEM_SHARED((n_pad, d), jnp.float32),
            pltpu.SemaphoreType.DMA,
            pltpu.SemaphoreType.DMA,
        ],
    )
    def agg(u_hbm, idx_hbm, out_hbm, idx_v, msg0, msg1, acc, gsem, ssem):
        cid = lax.axis_index("c")
        sid = lax.axis_index("s")
        msgs = (msg0, msg1)
        _fill_const(msg0, ZR, d, 0.0)
        base_r = sid * rows_per_tile
        for j in range(nzc):
            pltpu.sync_copy(msg0, acc.at[pl.ds(base_r + j * ZR, ZR)])
        plsc.subcore_barrier()

        my_w = jnp.where(cid == 0, w0, w1)
        c0 = cid * NS * w0 + sid * my_w
        pltpu.sync_copy(idx_hbm.at[c0], idx_v.at[0])
        pltpu.async_copy(u_hbm.at[idx_v.at[0, 0]], msg0, gsem)

        def body(t, carry):
            for p in range(2):
                i = 2 * t + p
                q = 1 - p

                @pl.when(i > 0)
                def _():
                    pltpu.make_async_copy(
                        msgs[q], acc.at[idx_v.at[q, 1]], ssem).wait()

                @pl.when(i + 1 < my_w)
                def _():
                    pltpu.sync_copy(idx_hbm.at[c0 + i + 1], idx_v.at[q])
                    pltpu.async_copy(u_hbm.at[idx_v.at[q, 0]], msgs[q], gsem)

                pltpu.make_async_copy(
                    u_hbm.at[idx_v.at[p, 0]], msgs[p], gsem).wait()
                pltpu.async_copy(msgs[p], acc.at[idx_v.at[p, 1]], ssem, add=True)
            return carry

        lax.fori_loop(0, my_w // 2, body, 0)
        pltpu.make_async_copy(msgs[1], acc.at[idx_v.at[1, 1]], ssem).wait()
        plsc.subcore_barrier()
        pltpu.sync_copy(acc.at[pl.ds(base_r, rows_per_tile)],
                        out_hbm.at[cid, pl.ds(base_r, rows_per_tile)])

    return agg


def _make_sc_deg(n, ep):
    d = NLANE
    n_chunks = ep // CH
    per_w = n_chunks // (NC * NS)
    rows_per_tile = _pad_rows(n)
    n_pad = NS * rows_per_tile
    nzc = rows_per_tile // ZR

    mesh = plsc.VectorSubcoreMesh(core_axis_name="c", subcore_axis_name="s")

    @functools.partial(
        pl.kernel,
        out_type=jax.ShapeDtypeStruct((NC, n_pad, d), jnp.float32),
        mesh=mesh,
        compiler_params=pltpu.CompilerParams(use_tc_tiling_on_sc=False),
        scratch_types=[
            pltpu.VMEM((2, 2, CH), jnp.int32),
            pltpu.VMEM((CH, d), jnp.float32),
            pltpu.VMEM((ZR, d), jnp.float32),
            pltpu.VMEM_SHARED((n_pad, d), jnp.float32),
            pltpu.SemaphoreType.DMA,
        ],
    )
    def deg(idx_hbm, out_hbm, idx_v, ones_v, zero_v, acc, ssem):
        cid = lax.axis_index("c")
        sid = lax.axis_index("s")
        wid = cid * NS + sid
        _fill_const(zero_v, ZR, d, 0.0)
        _fill_const(ones_v, CH, d, 1.0)
        base_r = sid * rows_per_tile
        for j in range(nzc):
            pltpu.sync_copy(zero_v, acc.at[pl.ds(base_r + j * ZR, ZR)])
        plsc.subcore_barrier()

        c0 = wid * per_w
        pltpu.sync_copy(idx_hbm.at[c0], idx_v.at[0])

        def body(t, carry):
            for p in range(2):
                i = 2 * t + p
                q = 1 - p

                @pl.when(i > 0)
                def _():
                    pltpu.make_async_copy(
                        ones_v, acc.at[idx_v.at[q, 1]], ssem).wait()

                pltpu.async_copy(ones_v, acc.at[idx_v.at[p, 1]], ssem, add=True)

                @pl.when(i + 1 < per_w)
                def _():
                    pltpu.sync_copy(idx_hbm.at[c0 + i + 1], idx_v.at[q])
            return carry

        lax.fori_loop(0, per_w // 2, body, 0)
        pltpu.make_async_copy(ones_v, acc.at[idx_v.at[1, 1]], ssem).wait()
        plsc.subcore_barrier()
        pltpu.sync_copy(acc.at[pl.ds(base_r, rows_per_tile)],
                        out_hbm.at[cid, pl.ds(base_r, rows_per_tile)])

    return deg


def _elu(x):
    return jnp.where(x > 0, x, jnp.exp(x) - 1.0)


def _prep_body(dg0, dg1, x, q_out, dinv_out):
    deg = dg0[0][:, 0:1] + dg1[0][:, 0:1] + 1.0
    dinv = lax.rsqrt(deg)
    q_out[...] = x[...] * dinv
    dinv_out[...] = jnp.broadcast_to(dinv, dinv_out.shape)


def _l1_body(a0, a1, q, dinv, W1, b1, fc1W, fc1b, h2_out, mom_out):
    t = a0[0] + a1[0] + q[...]
    h = _elu((jnp.dot(t, W1[...], preferred_element_type=jnp.float32)
              * dinv[:, 0:1]) + b1[...])
    h2 = _elu(jnp.dot(h, fc1W[...], preferred_element_type=jnp.float32) + fc1b[...])
    h2_out[...] = h2
    s1 = jnp.sum(h2, axis=0, keepdims=True)
    s2 = jnp.sum(h2 * h2, axis=0, keepdims=True)
    delta = jnp.concatenate([s1, s2], axis=0)

    @pl.when(pl.program_id(0) == 0)
    def _():
        mom_out[...] = jnp.zeros_like(mom_out)

    mom_out[...] += delta


def _make_l2_body(n):
    def _l2_body(h2, mom, dinv, W2, u2_out):
        m = mom[...]
        mean = m[0:1, :] * (1.0 / n)
        var = m[1:2, :] * (1.0 / n) - mean * mean
        s = lax.rsqrt(var + 1e-5)
        y = (h2[...] - mean) * s
        u2_out[...] = (jnp.dot(y, W2[...], preferred_element_type=jnp.float32)
                       * dinv[:, 0:1])
    return _l2_body


def _l2post_body(a0, a1, u2, dinv, b2, fc2W, fc2b, p3_out):
    dv = dinv[:, 0:1]
    y2 = _elu(dv * (a0[0] + a1[0] + u2[...]) + b2[...])
    y3 = _elu(jnp.dot(y2, fc2W[...], preferred_element_type=jnp.float32) + fc2b[...])
    p3_out[...] = y3 * dv


def _out_body(a0, a1, p3, dinv, W3p, b3p, fc3Wp, fc3bp, z_out):
    dv = dinv[:, 0:1]
    t3 = a0[0] + a1[0] + p3[...]
    z = _elu(dv * jnp.dot(t3, W3p[...], preferred_element_type=jnp.float32) + b3p[...])
    z_out[...] = jnp.dot(z, fc3Wp[...], preferred_element_type=jnp.float32) + fc3bp[...]


def _rows_spec(rb, dcol, offset_blocks=0):
    return pl.BlockSpec((rb, dcol), lambda r: (r + offset_blocks, 0))


def _part_spec(rb, dcol, c):
    return pl.BlockSpec((1, rb, dcol), lambda r: (c, r, 0))


def _full_spec(shape):
    return pl.BlockSpec(shape, lambda r: tuple(0 for _ in shape))


def kernel(x, adj, num_graphs, in_batch, cluster,
           W1, b1, fc1W, fc1b, W2, b2, fc2W, fc2b, W3, b3, fc3W, fc3b):
    n, f = x.shape
    e = adj.shape[1]
    d1 = W1.shape[1]
    d2 = W2.shape[1]
    d3p = NLANE

    epw = NC * NS * CH * 2
    ep = ((e + epw - 1) // epw) * epw
    row = adj[0]
    col = adj[1]
    if ep != e:
        spare = NS * _pad_rows(n) - n
        pad_ar = jnp.arange(ep - e, dtype=col.dtype)
        row = jnp.concatenate([row, pad_ar % n])
        col = jnp.concatenate([col, n + pad_ar % min(CH, spare)])
    packed = jnp.stack([row.reshape(-1, CH), col.reshape(-1, CH)], axis=1)

    rb = 1000
    grid = (n // rb,)

    degp = _make_sc_deg(n, ep)(packed)

    q, dinv = pl.pallas_call(
        _prep_body,
        grid=grid,
        in_specs=[_part_spec(rb, d3p, 0), _part_spec(rb, d3p, 1),
                  _rows_spec(rb, f)],
        out_specs=[_rows_spec(rb, f), _rows_spec(rb, 8)],
        out_shape=[jax.ShapeDtypeStruct((n, f), jnp.float32),
                   jax.ShapeDtypeStruct((n, 8), jnp.float32)],
    )(degp, degp, x)

    agg128 = _make_sc_agg(n, ep, f)
    a1p = agg128(q, packed)

    h2, mom = pl.pallas_call(
        _l1_body,
        grid=grid,
        in_specs=[_part_spec(rb, f, 0), _part_spec(rb, f, 1),
                  _rows_spec(rb, f), _rows_spec(rb, 8),
                  _full_spec(W1.shape), _full_spec((1, d1)),
                  _full_spec(fc1W.shape), _full_spec((1, d1))],
        out_specs=[_rows_spec(rb, d1), _full_spec((2, d1))],
        out_shape=[jax.ShapeDtypeStruct((n, d1), jnp.float32),
                   jax.ShapeDtypeStruct((2, d1), jnp.float32)],
    )(a1p, a1p, q, dinv, W1, b1.reshape(1, -1), fc1W, fc1b.reshape(1, -1))

    u2 = pl.pallas_call(
        _make_l2_body(float(n)),
        grid=grid,
        in_specs=[_rows_spec(rb, d1), _full_spec((2, d1)),
                  _rows_spec(rb, 8), _full_spec(W2.shape)],
        out_specs=_rows_spec(rb, d2),
        out_shape=jax.ShapeDtypeStruct((n, d2), jnp.float32),
    )(h2, mom, dinv, W2)

    a2p = agg128(u2, packed)

    p3 = pl.pallas_call(
        _l2post_body,
        grid=grid,
        in_specs=[_part_spec(rb, d2, 0), _part_spec(rb, d2, 1),
                  _rows_spec(rb, d2), _rows_spec(rb, 8),
                  _full_spec((1, d2)), _full_spec(fc2W.shape),
                  _full_spec((1, d2))],
        out_specs=_rows_spec(rb, d2),
        out_shape=jax.ShapeDtypeStruct((n, d2), jnp.float32),
    )(a2p, a2p, u2, dinv, b2.reshape(1, -1), fc2W, fc2b.reshape(1, -1))

    a3p = agg128(p3, packed)

    nout = W3.shape[1]
    W3p = jnp.zeros((W3.shape[0], d3p), jnp.float32).at[:, :nout].set(W3)
    b3p = jnp.zeros((1, d3p), jnp.float32).at[0, :nout].set(b3)
    fc3Wp = jnp.zeros((d3p, d3p), jnp.float32).at[:nout, :nout].set(fc3W)
    fc3bp = jnp.zeros((1, d3p), jnp.float32).at[0, :nout].set(fc3b)
    zp = pl.pallas_call(
        _out_body,
        grid=grid,
        in_specs=[_part_spec(rb, d2, 0), _part_spec(rb, d2, 1),
                  _rows_spec(rb, d2), _rows_spec(rb, 8),
                  _full_spec((W3.shape[0], d3p)), _full_spec((1, d3p)),
                  _full_spec((d3p, d3p)), _full_spec((1, d3p))],
        out_specs=_rows_spec(rb, d3p),
        out_shape=jax.ShapeDtypeStruct((n, d3p), jnp.float32),
    )(a3p, a3p, p3, dinv, W3p, b3p, fc3Wp, fc3bp)

    return zp[:, :nout]


# speedup vs baseline: 2.7682x; 1.3068x over previous
import functools

import jax
import jax.numpy as jnp
from jax import lax
from jax.experimental import pallas as pl
from jax.experimental.pallas import tpu as pltpu
from jax.experimental.pallas import tpu_sc as plsc

NC, NS, NLANE = 2, 16, 16
CH = 128
ZR = 128
_SC0_FRAC = 0.5


def _fill_const(ref, rows, cols, val):
    groups = cols // NLANE

    def body(i, carry):
        r = i // groups
        g = i % groups
        ref[r, pl.ds(g * NLANE, NLANE)] = jnp.full((NLANE,), val, jnp.float32)
        return carry

    lax.fori_loop(0, rows * groups, body, 0)


def _pad_rows(n):
    return ZR * ((n + NS * ZR - 1) // (NS * ZR))


def _make_sc_agg(n, ep, d):
    n_chunks = ep // CH
    per_sc = n_chunks // NC
    w0 = 2 * int(round(_SC0_FRAC * (n_chunks // NS) / 2.0))
    w1 = n_chunks // NS - w0
    rows_per_tile = _pad_rows(n)
    n_pad = NS * rows_per_tile
    nzc = rows_per_tile // ZR
    assert n_chunks % (NC * NS) == 0 and w0 % 2 == 0 and w1 % 2 == 0
    assert w0 >= 2 and w1 >= 2 and n_pad > n

    mesh = plsc.VectorSubcoreMesh(core_axis_name="c", subcore_axis_name="s")

    @functools.partial(
        pl.kernel,
        out_type=jax.ShapeDtypeStruct((NC, n_pad, d), jnp.float32),
        mesh=mesh,
        scratch_types=[
            pltpu.VMEM((2, 2, CH), jnp.int32),
            pltpu.VMEM((CH, d), jnp.float32),
            pltpu.VMEM((CH, d), jnp.float32),
            pltpu.VMEM_SHARED((n_pad, d), jnp.float32),
            pltpu.SemaphoreType.DMA,
            pltpu.SemaphoreType.DMA,
        ],
    )
    def agg(u_hbm, idx_hbm, out_hbm, idx_v, msg0, msg1, acc, gsem, ssem):
        cid = lax.axis_index("c")
        sid = lax.axis_index("s")
        msgs = (msg0, msg1)
        _fill_const(msg0, ZR, d, 0.0)
        base_r = sid * rows_per_tile
        for j in range(nzc):
            pltpu.sync_copy(msg0, acc.at[pl.ds(base_r + j * ZR, ZR)])
        plsc.subcore_barrier()

        my_w = jnp.where(cid == 0, w0, w1)
        c0 = cid * NS * w0 + sid * my_w
        pltpu.sync_copy(idx_hbm.at[c0], idx_v.at[0])
        pltpu.async_copy(u_hbm.at[idx_v.at[0, 0]], msg0, gsem)

        def body(t, carry):
            for p in range(2):
                i = 2 * t + p
                q = 1 - p

                @pl.when(i > 0)
                def _():
                    pltpu.make_async_copy(
                        msgs[q], acc.at[idx_v.at[q, 1]], ssem).wait()

                @pl.when(i + 1 < my_w)
                def _():
                    pltpu.sync_copy(idx_hbm.at[c0 + i + 1], idx_v.at[q])
                    pltpu.async_copy(u_hbm.at[idx_v.at[q, 0]], msgs[q], gsem)

                pltpu.make_async_copy(
                    u_hbm.at[idx_v.at[p, 0]], msgs[p], gsem).wait()
                pltpu.async_copy(msgs[p], acc.at[idx_v.at[p, 1]], ssem, add=True)
            return carry

        lax.fori_loop(0, my_w // 2, body, 0)
        pltpu.make_async_copy(msgs[1], acc.at[idx_v.at[1, 1]], ssem).wait()
        plsc.subcore_barrier()
        pltpu.sync_copy(acc.at[pl.ds(base_r, rows_per_tile)],
                        out_hbm.at[cid, pl.ds(base_r, rows_per_tile)])

    return agg


def _make_sc_deg(n, ep):
    d = NLANE
    n_chunks = ep // CH
    per_w = n_chunks // (NC * NS)
    rows_per_tile = _pad_rows(n)
    n_pad = NS * rows_per_tile
    nzc = rows_per_tile // ZR

    mesh = plsc.VectorSubcoreMesh(core_axis_name="c", subcore_axis_name="s")

    @functools.partial(
        pl.kernel,
        out_type=jax.ShapeDtypeStruct((NC, n_pad, d), jnp.float32),
        mesh=mesh,
        compiler_params=pltpu.CompilerParams(use_tc_tiling_on_sc=False),
        scratch_types=[
            pltpu.VMEM((2, 2, CH), jnp.int32),
            pltpu.VMEM((CH, d), jnp.float32),
            pltpu.VMEM((ZR, d), jnp.float32),
            pltpu.VMEM_SHARED((n_pad, d), jnp.float32),
            pltpu.SemaphoreType.DMA,
        ],
    )
    def deg(idx_hbm, out_hbm, idx_v, ones_v, zero_v, acc, ssem):
        cid = lax.axis_index("c")
        sid = lax.axis_index("s")
        wid = cid * NS + sid
        _fill_const(zero_v, ZR, d, 0.0)
        _fill_const(ones_v, CH, d, 1.0)
        base_r = sid * rows_per_tile
        for j in range(nzc):
            pltpu.sync_copy(zero_v, acc.at[pl.ds(base_r + j * ZR, ZR)])
        plsc.subcore_barrier()

        c0 = wid * per_w
        pltpu.sync_copy(idx_hbm.at[c0], idx_v.at[0])

        def body(t, carry):
            for p in range(2):
                i = 2 * t + p
                q = 1 - p

                @pl.when(i > 0)
                def _():
                    pltpu.make_async_copy(
                        ones_v, acc.at[idx_v.at[q, 1]], ssem).wait()

                pltpu.async_copy(ones_v, acc.at[idx_v.at[p, 1]], ssem, add=True)

                @pl.when(i + 1 < per_w)
                def _():
                    pltpu.sync_copy(idx_hbm.at[c0 + i + 1], idx_v.at[q])
            return carry

        lax.fori_loop(0, per_w // 2, body, 0)
        pltpu.make_async_copy(ones_v, acc.at[idx_v.at[1, 1]], ssem).wait()
        plsc.subcore_barrier()
        pltpu.sync_copy(acc.at[pl.ds(base_r, rows_per_tile)],
                        out_hbm.at[cid, pl.ds(base_r, rows_per_tile)])

    return deg


def _elu(x):
    return jnp.where(x > 0, x, jnp.exp(x) - 1.0)


def _prep_body(dg0, dg1, x, q_out, dinv_out):
    deg = dg0[0][:, 0:1] + dg1[0][:, 0:1] + 1.0
    dinv = lax.rsqrt(deg)
    q_out[...] = x[...] * dinv
    dinv_out[...] = jnp.broadcast_to(dinv, dinv_out.shape)


def _l1_body(a0, a1, q, dinv, W1, b1, fc1W, fc1b, h2_out, mom_out):
    t = a0[0] + a1[0] + q[...]
    h = _elu((jnp.dot(t, W1[...], preferred_element_type=jnp.float32)
              * dinv[:, 0:1]) + b1[...])
    h2 = _elu(jnp.dot(h, fc1W[...], preferred_element_type=jnp.float32) + fc1b[...])
    h2_out[...] = h2
    s1 = jnp.sum(h2, axis=0, keepdims=True)
    s2 = jnp.sum(h2 * h2, axis=0, keepdims=True)
    delta = jnp.concatenate([s1, s2], axis=0)

    @pl.when(pl.program_id(0) == 0)
    def _():
        mom_out[...] = jnp.zeros_like(mom_out)

    mom_out[...] += delta


def _make_l2_body(n):
    def _l2_body(h2, mom, dinv, W2, u2_out):
        m = mom[...]
        mean = m[0:1, :] * (1.0 / n)
        var = m[1:2, :] * (1.0 / n) - mean * mean
        s = lax.rsqrt(var + 1e-5)
        y = (h2[...] - mean) * s
        u2_out[...] = (jnp.dot(y, W2[...], preferred_element_type=jnp.float32)
                       * dinv[:, 0:1])
    return _l2_body


def _l2post_body(a0, a1, u2, dinv, b2, fc2W, fc2b, p3_out):
    dv = dinv[:, 0:1]
    y2 = _elu(dv * (a0[0] + a1[0] + u2[...]) + b2[...])
    y3 = _elu(jnp.dot(y2, fc2W[...], preferred_element_type=jnp.float32) + fc2b[...])
    p3_out[...] = y3 * dv


def _out_body(a0, a1, p3, dinv, W3p, b3p, fc3Wp, fc3bp, z_out):
    dv = dinv[:, 0:1]
    t3 = a0[0] + a1[0] + p3[...]
    z = _elu(dv * jnp.dot(t3, W3p[...], preferred_element_type=jnp.float32) + b3p[...])
    z_out[...] = jnp.dot(z, fc3Wp[...], preferred_element_type=jnp.float32) + fc3bp[...]


def _rows_spec(rb, dcol, offset_blocks=0):
    return pl.BlockSpec((rb, dcol), lambda r: (r + offset_blocks, 0))


def _part_spec(rb, dcol, c):
    return pl.BlockSpec((1, rb, dcol), lambda r: (c, r, 0))


def _full_spec(shape):
    return pl.BlockSpec(shape, lambda r: tuple(0 for _ in shape))


def kernel(x, adj, num_graphs, in_batch, cluster,
           W1, b1, fc1W, fc1b, W2, b2, fc2W, fc2b, W3, b3, fc3W, fc3b):
    n, f = x.shape
    e = adj.shape[1]
    d1 = W1.shape[1]
    d2 = W2.shape[1]
    d3p = NLANE

    epw = NC * NS * CH * 2
    ep = ((e + epw - 1) // epw) * epw
    row = adj[0]
    col = adj[1]
    if ep != e:
        spare = NS * _pad_rows(n) - n
        pad_ar = jnp.arange(ep - e, dtype=col.dtype)
        row = jnp.concatenate([row, pad_ar % n])
        col = jnp.concatenate([col, n + pad_ar % min(CH, spare)])
    packed = jnp.stack([row.reshape(-1, CH), col.reshape(-1, CH)], axis=1)

    rb = 1000
    grid = (n // rb,)

    degp = _make_sc_deg(n, ep)(packed)

    q, dinv = pl.pallas_call(
        _prep_body,
        grid=grid,
        in_specs=[_part_spec(rb, d3p, 0), _part_spec(rb, d3p, 1),
                  _rows_spec(rb, f)],
        out_specs=[_rows_spec(rb, f), _rows_spec(rb, 8)],
        out_shape=[jax.ShapeDtypeStruct((n, f), jnp.float32),
                   jax.ShapeDtypeStruct((n, 8), jnp.float32)],
    )(degp, degp, x)

    agg128 = _make_sc_agg(n, ep, f)
    a1p = agg128(q, packed)

    h2, mom = pl.pallas_call(
        _l1_body,
        grid=grid,
        in_specs=[_part_spec(rb, f, 0), _part_spec(rb, f, 1),
                  _rows_spec(rb, f), _rows_spec(rb, 8),
                  _full_spec(W1.shape), _full_spec((1, d1)),
                  _full_spec(fc1W.shape), _full_spec((1, d1))],
        out_specs=[_rows_spec(rb, d1), _full_spec((2, d1))],
        out_shape=[jax.ShapeDtypeStruct((n, d1), jnp.float32),
                   jax.ShapeDtypeStruct((2, d1), jnp.float32)],
    )(a1p, a1p, q, dinv, W1, b1.reshape(1, -1), fc1W, fc1b.reshape(1, -1))

    u2 = pl.pallas_call(
        _make_l2_body(float(n)),
        grid=grid,
        in_specs=[_rows_spec(rb, d1), _full_spec((2, d1)),
                  _rows_spec(rb, 8), _full_spec(W2.shape)],
        out_specs=_rows_spec(rb, d2),
        out_shape=jax.ShapeDtypeStruct((n, d2), jnp.float32),
    )(h2, mom, dinv, W2)

    a2p = agg128(u2, packed)

    p3 = pl.pallas_call(
        _l2post_body,
        grid=grid,
        in_specs=[_part_spec(rb, d2, 0), _part_spec(rb, d2, 1),
                  _rows_spec(rb, d2), _rows_spec(rb, 8),
                  _full_spec((1, d2)), _full_spec(fc2W.shape),
                  _full_spec((1, d2))],
        out_specs=_rows_spec(rb, d2),
        out_shape=jax.ShapeDtypeStruct((n, d2), jnp.float32),
    )(a2p, a2p, u2, dinv, b2.reshape(1, -1), fc2W, fc2b.reshape(1, -1))

    a3p = agg128(p3, packed)

    nout = W3.shape[1]
    W3p = jnp.zeros((W3.shape[0], d3p), jnp.float32).at[:, :nout].set(W3)
    b3p = jnp.zeros((1, d3p), jnp.float32).at[0, :nout].set(b3)
    fc3Wp = jnp.zeros((d3p, d3p), jnp.float32).at[:nout, :nout].set(fc3W)
    fc3bp = jnp.zeros((1, d3p), jnp.float32).at[0, :nout].set(fc3b)
    zp = pl.pallas_call(
        _out_body,
        grid=grid,
        in_specs=[_part_spec(rb, d2, 0), _part_spec(rb, d2, 1),
                  _rows_spec(rb, d2), _rows_spec(rb, 8),
                  _full_spec((W3.shape[0], d3p)), _full_spec((1, d3p)),
                  _full_spec((d3p, d3p)), _full_spec((1, d3p))],
        out_specs=_rows_spec(rb, d3p),
        out_shape=jax.ShapeDtypeStruct((n, d3p), jnp.float32),
    )(a3p, a3p, p3, dinv, W3p, b3p, fc3Wp, fc3bp)

    return zp[:, :nout]


# speedup vs baseline: 2.9372x; 1.0611x over previous
import functools

import jax
import jax.numpy as jnp
from jax import lax
from jax.experimental import pallas as pl
from jax.experimental.pallas import tpu as pltpu
from jax.experimental.pallas import tpu_sc as plsc

NC, NS, NLANE = 2, 16, 16
CH = 128
ZR = 128
_SC0_FRAC = 0.5


def _fill_const(ref, rows, cols, val):
    groups = cols // NLANE

    def body(i, carry):
        r = i // groups
        g = i % groups
        ref[r, pl.ds(g * NLANE, NLANE)] = jnp.full((NLANE,), val, jnp.float32)
        return carry

    lax.fori_loop(0, rows * groups, body, 0)


def _pad_rows(n):
    return ZR * ((n + NS * ZR - 1) // (NS * ZR))


def _make_sc_agg(n, ep, d, tc_tiling=True):
    n_chunks = ep // CH
    per_sc = n_chunks // NC
    w0 = 2 * int(round(_SC0_FRAC * (n_chunks // NS) / 2.0))
    w1 = n_chunks // NS - w0
    rows_per_tile = _pad_rows(n)
    n_pad = NS * rows_per_tile
    nzc = rows_per_tile // ZR
    assert n_chunks % (NC * NS) == 0 and w0 % 2 == 0 and w1 % 2 == 0
    assert w0 >= 2 and w1 >= 2 and n_pad > n

    mesh = plsc.VectorSubcoreMesh(core_axis_name="c", subcore_axis_name="s")

    @functools.partial(
        pl.kernel,
        out_type=jax.ShapeDtypeStruct((NC, n_pad, d), jnp.float32),
        mesh=mesh,
        compiler_params=pltpu.CompilerParams(use_tc_tiling_on_sc=tc_tiling),
        scratch_types=[
            pltpu.VMEM((2, 2, CH), jnp.int32),
            pltpu.VMEM((CH, d), jnp.float32),
            pltpu.VMEM((CH, d), jnp.float32),
            pltpu.VMEM_SHARED((n_pad, d), jnp.float32),
            pltpu.SemaphoreType.DMA,
            pltpu.SemaphoreType.DMA,
        ],
    )
    def agg(u_hbm, idx_hbm, out_hbm, idx_v, msg0, msg1, acc, gsem, ssem):
        cid = lax.axis_index("c")
        sid = lax.axis_index("s")
        msgs = (msg0, msg1)
        _fill_const(msg0, ZR, d, 0.0)
        base_r = sid * rows_per_tile
        for j in range(nzc):
            pltpu.sync_copy(msg0, acc.at[pl.ds(base_r + j * ZR, ZR)])
        plsc.subcore_barrier()

        my_w = jnp.where(cid == 0, w0, w1)
        c0 = cid * NS * w0 + sid * my_w
        pltpu.sync_copy(idx_hbm.at[c0], idx_v.at[0])
        pltpu.async_copy(u_hbm.at[idx_v.at[0, 0]], msg0, gsem)

        def body(t, carry):
            for p in range(2):
                i = 2 * t + p
                q = 1 - p

                @pl.when(i > 0)
                def _():
                    pltpu.make_async_copy(
                        msgs[q], acc.at[idx_v.at[q, 1]], ssem).wait()

                @pl.when(i + 1 < my_w)
                def _():
                    pltpu.sync_copy(idx_hbm.at[c0 + i + 1], idx_v.at[q])
                    pltpu.async_copy(u_hbm.at[idx_v.at[q, 0]], msgs[q], gsem)

                pltpu.make_async_copy(
                    u_hbm.at[idx_v.at[p, 0]], msgs[p], gsem).wait()
                pltpu.async_copy(msgs[p], acc.at[idx_v.at[p, 1]], ssem, add=True)
            return carry

        lax.fori_loop(0, my_w // 2, body, 0)
        pltpu.make_async_copy(msgs[1], acc.at[idx_v.at[1, 1]], ssem).wait()
        plsc.subcore_barrier()
        pltpu.sync_copy(acc.at[pl.ds(base_r, rows_per_tile)],
                        out_hbm.at[cid, pl.ds(base_r, rows_per_tile)])

    return agg


def _make_sc_deg(n, ep):
    d = NLANE
    n_chunks = ep // CH
    per_w = n_chunks // (NC * NS)
    rows_per_tile = _pad_rows(n)
    n_pad = NS * rows_per_tile
    nzc = rows_per_tile // ZR

    mesh = plsc.VectorSubcoreMesh(core_axis_name="c", subcore_axis_name="s")

    @functools.partial(
        pl.kernel,
        out_type=jax.ShapeDtypeStruct((NC, n_pad, d), jnp.float32),
        mesh=mesh,
        compiler_params=pltpu.CompilerParams(use_tc_tiling_on_sc=False),
        scratch_types=[
            pltpu.VMEM((2, 2, CH), jnp.int32),
            pltpu.VMEM((CH, d), jnp.float32),
            pltpu.VMEM((ZR, d), jnp.float32),
            pltpu.VMEM_SHARED((n_pad, d), jnp.float32),
            pltpu.SemaphoreType.DMA,
        ],
    )
    def deg(idx_hbm, out_hbm, idx_v, ones_v, zero_v, acc, ssem):
        cid = lax.axis_index("c")
        sid = lax.axis_index("s")
        wid = cid * NS + sid
        _fill_const(zero_v, ZR, d, 0.0)
        _fill_const(ones_v, CH, d, 1.0)
        base_r = sid * rows_per_tile
        for j in range(nzc):
            pltpu.sync_copy(zero_v, acc.at[pl.ds(base_r + j * ZR, ZR)])
        plsc.subcore_barrier()

        c0 = wid * per_w
        pltpu.sync_copy(idx_hbm.at[c0], idx_v.at[0])

        def body(t, carry):
            for p in range(2):
                i = 2 * t + p
                q = 1 - p

                @pl.when(i > 0)
                def _():
                    pltpu.make_async_copy(
                        ones_v, acc.at[idx_v.at[q, 1]], ssem).wait()

                pltpu.async_copy(ones_v, acc.at[idx_v.at[p, 1]], ssem, add=True)

                @pl.when(i + 1 < per_w)
                def _():
                    pltpu.sync_copy(idx_hbm.at[c0 + i + 1], idx_v.at[q])
            return carry

        lax.fori_loop(0, per_w // 2, body, 0)
        pltpu.make_async_copy(ones_v, acc.at[idx_v.at[1, 1]], ssem).wait()
        plsc.subcore_barrier()
        pltpu.sync_copy(acc.at[pl.ds(base_r, rows_per_tile)],
                        out_hbm.at[cid, pl.ds(base_r, rows_per_tile)])

    return deg


def _elu(x):
    return jnp.where(x > 0, x, jnp.exp(x) - 1.0)


def _prep_body(dg0, dg1, x, q_out, dinv_out):
    deg = dg0[0][:, 0:1] + dg1[0][:, 0:1] + 1.0
    dinv = lax.rsqrt(deg)
    q_out[...] = x[...] * dinv
    dinv_out[...] = jnp.broadcast_to(dinv, dinv_out.shape)


def _l1_body(a0, a1, q, dinv, W1, b1, fc1W, fc1b, h2_out, mom_out):
    t = a0[0] + a1[0] + q[...]
    h = _elu((jnp.dot(t, W1[...], preferred_element_type=jnp.float32)
              * dinv[:, 0:1]) + b1[...])
    h2 = _elu(jnp.dot(h, fc1W[...], preferred_element_type=jnp.float32) + fc1b[...])
    h2_out[...] = h2
    s1 = jnp.sum(h2, axis=0, keepdims=True)
    s2 = jnp.sum(h2 * h2, axis=0, keepdims=True)
    delta = jnp.concatenate([s1, s2], axis=0)

    @pl.when(pl.program_id(0) == 0)
    def _():
        mom_out[...] = jnp.zeros_like(mom_out)

    mom_out[...] += delta


def _make_l2_body(n):
    def _l2_body(h2, mom, dinv, W2, u2_out):
        m = mom[...]
        mean = m[0:1, :] * (1.0 / n)
        var = m[1:2, :] * (1.0 / n) - mean * mean
        s = lax.rsqrt(var + 1e-5)
        y = (h2[...] - mean) * s
        u2_out[...] = (jnp.dot(y, W2[...], preferred_element_type=jnp.float32)
                       * dinv[:, 0:1])
    return _l2_body


def _l2post_body(a0, a1, u2, dinv, b2, fc2W, fc2b, W3p, u3_out):
    dv = dinv[:, 0:1]
    y2 = _elu(dv * (a0[0] + a1[0] + u2[...]) + b2[...])
    y3 = _elu(jnp.dot(y2, fc2W[...], preferred_element_type=jnp.float32) + fc2b[...])
    u3_out[...] = jnp.dot(y3, W3p[...], preferred_element_type=jnp.float32) * dv


def _out_body(a0, a1, u3, dinv, b3p, fc3Wp, fc3bp, z_out):
    dv = dinv[:, 0:1]
    z = _elu(dv * (a0[0] + a1[0] + u3[...]) + b3p[...])
    z_out[...] = jnp.dot(z, fc3Wp[...], preferred_element_type=jnp.float32) + fc3bp[...]


def _rows_spec(rb, dcol, offset_blocks=0):
    return pl.BlockSpec((rb, dcol), lambda r: (r + offset_blocks, 0))


def _part_spec(rb, dcol, c):
    return pl.BlockSpec((1, rb, dcol), lambda r: (c, r, 0))


def _full_spec(shape):
    return pl.BlockSpec(shape, lambda r: tuple(0 for _ in shape))


def kernel(x, adj, num_graphs, in_batch, cluster,
           W1, b1, fc1W, fc1b, W2, b2, fc2W, fc2b, W3, b3, fc3W, fc3b):
    n, f = x.shape
    e = adj.shape[1]
    d1 = W1.shape[1]
    d2 = W2.shape[1]
    d3p = NLANE

    epw = NC * NS * CH * 2
    ep = ((e + epw - 1) // epw) * epw
    row = adj[0]
    col = adj[1]
    if ep != e:
        spare = NS * _pad_rows(n) - n
        pad_ar = jnp.arange(ep - e, dtype=col.dtype)
        row = jnp.concatenate([row, pad_ar % n])
        col = jnp.concatenate([col, n + pad_ar % min(CH, spare)])
    packed = jnp.stack([row.reshape(-1, CH), col.reshape(-1, CH)], axis=1)

    rb = 1000
    grid = (n // rb,)

    degp = _make_sc_deg(n, ep)(packed)

    q, dinv = pl.pallas_call(
        _prep_body,
        grid=grid,
        in_specs=[_part_spec(rb, d3p, 0), _part_spec(rb, d3p, 1),
                  _rows_spec(rb, f)],
        out_specs=[_rows_spec(rb, f), _rows_spec(rb, 8)],
        out_shape=[jax.ShapeDtypeStruct((n, f), jnp.float32),
                   jax.ShapeDtypeStruct((n, 8), jnp.float32)],
    )(degp, degp, x)

    agg128 = _make_sc_agg(n, ep, f)
    a1p = agg128(q, packed)

    h2, mom = pl.pallas_call(
        _l1_body,
        grid=grid,
        in_specs=[_part_spec(rb, f, 0), _part_spec(rb, f, 1),
                  _rows_spec(rb, f), _rows_spec(rb, 8),
                  _full_spec(W1.shape), _full_spec((1, d1)),
                  _full_spec(fc1W.shape), _full_spec((1, d1))],
        out_specs=[_rows_spec(rb, d1), _full_spec((2, d1))],
        out_shape=[jax.ShapeDtypeStruct((n, d1), jnp.float32),
                   jax.ShapeDtypeStruct((2, d1), jnp.float32)],
    )(a1p, a1p, q, dinv, W1, b1.reshape(1, -1), fc1W, fc1b.reshape(1, -1))

    u2 = pl.pallas_call(
        _make_l2_body(float(n)),
        grid=grid,
        in_specs=[_rows_spec(rb, d1), _full_spec((2, d1)),
                  _rows_spec(rb, 8), _full_spec(W2.shape)],
        out_specs=_rows_spec(rb, d2),
        out_shape=jax.ShapeDtypeStruct((n, d2), jnp.float32),
    )(h2, mom, dinv, W2)

    a2p = agg128(u2, packed)

    nout = W3.shape[1]
    W3p = jnp.zeros((W3.shape[0], d3p), jnp.float32).at[:, :nout].set(W3)
    u3 = pl.pallas_call(
        _l2post_body,
        grid=grid,
        in_specs=[_part_spec(rb, d2, 0), _part_spec(rb, d2, 1),
                  _rows_spec(rb, d2), _rows_spec(rb, 8),
                  _full_spec((1, d2)), _full_spec(fc2W.shape),
                  _full_spec((1, d2)), _full_spec((W3.shape[0], d3p))],
        out_specs=_rows_spec(rb, d3p),
        out_shape=jax.ShapeDtypeStruct((n, d3p), jnp.float32),
    )(a2p, a2p, u2, dinv, b2.reshape(1, -1), fc2W, fc2b.reshape(1, -1), W3p)

    a3p = _make_sc_agg(n, ep, d3p, tc_tiling=False)(u3, packed)

    b3p = jnp.zeros((1, d3p), jnp.float32).at[0, :nout].set(b3)
    fc3Wp = jnp.zeros((d3p, d3p), jnp.float32).at[:nout, :nout].set(fc3W)
    fc3bp = jnp.zeros((1, d3p), jnp.float32).at[0, :nout].set(fc3b)
    zp = pl.pallas_call(
        _out_body,
        grid=grid,
        in_specs=[_part_spec(rb, d3p, 0), _part_spec(rb, d3p, 1),
                  _rows_spec(rb, d3p), _rows_spec(rb, 8),
                  _full_spec((1, d3p)), _full_spec((d3p, d3p)),
                  _full_spec((1, d3p))],
        out_specs=_rows_spec(rb, d3p),
        out_shape=jax.ShapeDtypeStruct((n, d3p), jnp.float32),
    )(a3p, a3p, u3, dinv, b3p, fc3Wp, fc3bp)

    return zp[:, :nout]


# speedup vs baseline: 2.9937x; 1.0192x over previous
import functools

import jax
import jax.numpy as jnp
from jax import lax
from jax.experimental import pallas as pl
from jax.experimental.pallas import tpu as pltpu
from jax.experimental.pallas import tpu_sc as plsc

NC, NS, NLANE = 2, 16, 16
CH = 128
ZR = 128
_SC0_FRAC = 0.5


def _fill_const(ref, rows, cols, val):
    groups = cols // NLANE

    def body(i, carry):
        r = i // groups
        g = i % groups
        ref[r, pl.ds(g * NLANE, NLANE)] = jnp.full((NLANE,), val, jnp.float32)
        return carry

    lax.fori_loop(0, rows * groups, body, 0)


def _pad_rows(n):
    return ZR * ((n + NS * ZR - 1) // (NS * ZR))


def _make_sc_agg(n, ep, d, tc_tiling=True):
    n_chunks = ep // CH
    per_sc = n_chunks // NC
    w0 = 2 * int(round(_SC0_FRAC * (n_chunks // NS) / 2.0))
    w1 = n_chunks // NS - w0
    rows_per_tile = _pad_rows(n)
    n_pad = NS * rows_per_tile
    nzc = rows_per_tile // ZR
    nbuf = 2 if d >= 128 else 4
    assert n_chunks % (NC * NS) == 0 and w0 % nbuf == 0 and w1 % nbuf == 0
    assert w0 >= nbuf and w1 >= nbuf and n_pad > n

    mesh = plsc.VectorSubcoreMesh(core_axis_name="c", subcore_axis_name="s")

    @functools.partial(
        pl.kernel,
        out_type=jax.ShapeDtypeStruct((NC, n_pad, d), jnp.float32),
        mesh=mesh,
        compiler_params=pltpu.CompilerParams(use_tc_tiling_on_sc=tc_tiling),
        scratch_types=[
            pltpu.VMEM((nbuf, 2, CH), jnp.int32),
        ] + [pltpu.VMEM((CH, d), jnp.float32) for _ in range(nbuf)] + [
            pltpu.VMEM_SHARED((n_pad, d), jnp.float32),
            pltpu.SemaphoreType.DMA,
            pltpu.SemaphoreType.DMA,
        ],
    )
    def agg(u_hbm, idx_hbm, out_hbm, idx_v, *rest):
        msgs = rest[:nbuf]
        acc, gsem, ssem = rest[nbuf:]
        cid = lax.axis_index("c")
        sid = lax.axis_index("s")
        _fill_const(msgs[0], ZR, d, 0.0)
        base_r = sid * rows_per_tile
        for j in range(nzc):
            pltpu.sync_copy(msgs[0], acc.at[pl.ds(base_r + j * ZR, ZR)])
        plsc.subcore_barrier()

        my_w = jnp.where(cid == 0, w0, w1)
        c0 = cid * NS * w0 + sid * my_w
        pltpu.sync_copy(idx_hbm.at[c0], idx_v.at[0])
        pltpu.async_copy(u_hbm.at[idx_v.at[0, 0]], msgs[0], gsem)

        def body(t, carry):
            for p in range(nbuf):
                i = nbuf * t + p
                q = (p + 1) % nbuf

                @pl.when(i >= nbuf - 1)
                def _():
                    pltpu.make_async_copy(
                        msgs[q], acc.at[idx_v.at[q, 1]], ssem).wait()

                @pl.when(i + 1 < my_w)
                def _():
                    pltpu.sync_copy(idx_hbm.at[c0 + i + 1], idx_v.at[q])
                    pltpu.async_copy(u_hbm.at[idx_v.at[q, 0]], msgs[q], gsem)

                pltpu.make_async_copy(
                    u_hbm.at[idx_v.at[p, 0]], msgs[p], gsem).wait()
                pltpu.async_copy(msgs[p], acc.at[idx_v.at[p, 1]], ssem, add=True)
            return carry

        lax.fori_loop(0, my_w // nbuf, body, 0)
        for j in range(nbuf - 1):
            pltpu.make_async_copy(msgs[j], acc.at[idx_v.at[j, 1]], ssem).wait()
        plsc.subcore_barrier()
        pltpu.sync_copy(acc.at[pl.ds(base_r, rows_per_tile)],
                        out_hbm.at[cid, pl.ds(base_r, rows_per_tile)])

    return agg


def _make_sc_deg(n, ep):
    d = NLANE
    n_chunks = ep // CH
    per_w = n_chunks // (NC * NS)
    rows_per_tile = _pad_rows(n)
    n_pad = NS * rows_per_tile
    nzc = rows_per_tile // ZR

    mesh = plsc.VectorSubcoreMesh(core_axis_name="c", subcore_axis_name="s")

    @functools.partial(
        pl.kernel,
        out_type=jax.ShapeDtypeStruct((NC, n_pad, d), jnp.float32),
        mesh=mesh,
        compiler_params=pltpu.CompilerParams(use_tc_tiling_on_sc=False),
        scratch_types=[
            pltpu.VMEM((4, 2, CH), jnp.int32),
            pltpu.VMEM((CH, d), jnp.float32),
            pltpu.VMEM((ZR, d), jnp.float32),
            pltpu.VMEM_SHARED((n_pad, d), jnp.float32),
            pltpu.SemaphoreType.DMA,
        ],
    )
    def deg(idx_hbm, out_hbm, idx_v, ones_v, zero_v, acc, ssem):
        cid = lax.axis_index("c")
        sid = lax.axis_index("s")
        wid = cid * NS + sid
        _fill_const(zero_v, ZR, d, 0.0)
        _fill_const(ones_v, CH, d, 1.0)
        base_r = sid * rows_per_tile
        for j in range(nzc):
            pltpu.sync_copy(zero_v, acc.at[pl.ds(base_r + j * ZR, ZR)])
        plsc.subcore_barrier()

        c0 = wid * per_w
        pltpu.sync_copy(idx_hbm.at[c0], idx_v.at[0])

        def body(t, carry):
            for p in range(4):
                i = 4 * t + p
                q = (p + 1) % 4

                @pl.when(i >= 3)
                def _():
                    pltpu.make_async_copy(
                        ones_v, acc.at[idx_v.at[q, 1]], ssem).wait()

                pltpu.async_copy(ones_v, acc.at[idx_v.at[p, 1]], ssem, add=True)

                @pl.when(i + 1 < per_w)
                def _():
                    pltpu.sync_copy(idx_hbm.at[c0 + i + 1], idx_v.at[q])
            return carry

        lax.fori_loop(0, per_w // 4, body, 0)
        for j in range(3):
            pltpu.make_async_copy(ones_v, acc.at[idx_v.at[j, 1]], ssem).wait()
        plsc.subcore_barrier()
        pltpu.sync_copy(acc.at[pl.ds(base_r, rows_per_tile)],
                        out_hbm.at[cid, pl.ds(base_r, rows_per_tile)])

    return deg


def _elu(x):
    return jnp.where(x > 0, x, jnp.exp(x) - 1.0)


def _prep_body(dg0, dg1, x, q_out, dinv_out):
    deg = dg0[0][:, 0:1] + dg1[0][:, 0:1] + 1.0
    dinv = lax.rsqrt(deg)
    q_out[...] = x[...] * dinv
    dinv_out[...] = jnp.broadcast_to(dinv, dinv_out.shape)


def _l1_body(a0, a1, q, dinv, W1, b1, fc1W, fc1b, h2_out, mom_out):
    t = a0[0] + a1[0] + q[...]
    h = _elu((jnp.dot(t, W1[...], preferred_element_type=jnp.float32)
              * dinv[:, 0:1]) + b1[...])
    h2 = _elu(jnp.dot(h, fc1W[...], preferred_element_type=jnp.float32) + fc1b[...])
    h2_out[...] = h2
    s1 = jnp.sum(h2, axis=0, keepdims=True)
    s2 = jnp.sum(h2 * h2, axis=0, keepdims=True)
    delta = jnp.concatenate([s1, s2], axis=0)

    @pl.when(pl.program_id(0) == 0)
    def _():
        mom_out[...] = jnp.zeros_like(mom_out)

    mom_out[...] += delta


def _make_l2_body(n):
    def _l2_body(h2, mom, dinv, W2, u2_out):
        m = mom[...]
        mean = m[0:1, :] * (1.0 / n)
        var = m[1:2, :] * (1.0 / n) - mean * mean
        s = lax.rsqrt(var + 1e-5)
        y = (h2[...] - mean) * s
        u2_out[...] = (jnp.dot(y, W2[...], preferred_element_type=jnp.float32)
                       * dinv[:, 0:1])
    return _l2_body


def _l2post_body(a0, a1, u2, dinv, b2, fc2W, fc2b, W3p, u3_out):
    dv = dinv[:, 0:1]
    y2 = _elu(dv * (a0[0] + a1[0] + u2[...]) + b2[...])
    y3 = _elu(jnp.dot(y2, fc2W[...], preferred_element_type=jnp.float32) + fc2b[...])
    u3_out[...] = jnp.dot(y3, W3p[...], preferred_element_type=jnp.float32) * dv


def _out_body(a0, a1, u3, dinv, b3p, fc3Wp, fc3bp, z_out):
    dv = dinv[:, 0:1]
    z = _elu(dv * (a0[0] + a1[0] + u3[...]) + b3p[...])
    z_out[...] = jnp.dot(z, fc3Wp[...], preferred_element_type=jnp.float32) + fc3bp[...]


def _rows_spec(rb, dcol, offset_blocks=0):
    return pl.BlockSpec((rb, dcol), lambda r: (r + offset_blocks, 0))


def _part_spec(rb, dcol, c):
    return pl.BlockSpec((1, rb, dcol), lambda r: (c, r, 0))


def _full_spec(shape):
    return pl.BlockSpec(shape, lambda r: tuple(0 for _ in shape))


def kernel(x, adj, num_graphs, in_batch, cluster,
           W1, b1, fc1W, fc1b, W2, b2, fc2W, fc2b, W3, b3, fc3W, fc3b):
    n, f = x.shape
    e = adj.shape[1]
    d1 = W1.shape[1]
    d2 = W2.shape[1]
    d3p = NLANE

    epw = NC * NS * CH * 2
    ep = ((e + epw - 1) // epw) * epw
    row = adj[0]
    col = adj[1]
    if ep != e:
        spare = NS * _pad_rows(n) - n
        pad_ar = jnp.arange(ep - e, dtype=col.dtype)
        row = jnp.concatenate([row, pad_ar % n])
        col = jnp.concatenate([col, n + pad_ar % min(CH, spare)])
    packed = jnp.stack([row.reshape(-1, CH), col.reshape(-1, CH)], axis=1)

    rb = 1000
    grid = (n // rb,)

    degp = _make_sc_deg(n, ep)(packed)

    q, dinv = pl.pallas_call(
        _prep_body,
        grid=grid,
        in_specs=[_part_spec(rb, d3p, 0), _part_spec(rb, d3p, 1),
                  _rows_spec(rb, f)],
        out_specs=[_rows_spec(rb, f), _rows_spec(rb, 8)],
        out_shape=[jax.ShapeDtypeStruct((n, f), jnp.float32),
                   jax.ShapeDtypeStruct((n, 8), jnp.float32)],
    )(degp, degp, x)

    agg128 = _make_sc_agg(n, ep, f)
    a1p = agg128(q, packed)

    h2, mom = pl.pallas_call(
        _l1_body,
        grid=grid,
        in_specs=[_part_spec(rb, f, 0), _part_spec(rb, f, 1),
                  _rows_spec(rb, f), _rows_spec(rb, 8),
                  _full_spec(W1.shape), _full_spec((1, d1)),
                  _full_spec(fc1W.shape), _full_spec((1, d1))],
        out_specs=[_rows_spec(rb, d1), _full_spec((2, d1))],
        out_shape=[jax.ShapeDtypeStruct((n, d1), jnp.float32),
                   jax.ShapeDtypeStruct((2, d1), jnp.float32)],
    )(a1p, a1p, q, dinv, W1, b1.reshape(1, -1), fc1W, fc1b.reshape(1, -1))

    u2 = pl.pallas_call(
        _make_l2_body(float(n)),
        grid=grid,
        in_specs=[_rows_spec(rb, d1), _full_spec((2, d1)),
                  _rows_spec(rb, 8), _full_spec(W2.shape)],
        out_specs=_rows_spec(rb, d2),
        out_shape=jax.ShapeDtypeStruct((n, d2), jnp.float32),
    )(h2, mom, dinv, W2)

    a2p = agg128(u2, packed)

    nout = W3.shape[1]
    W3p = jnp.zeros((W3.shape[0], d3p), jnp.float32).at[:, :nout].set(W3)
    u3 = pl.pallas_call(
        _l2post_body,
        grid=grid,
        in_specs=[_part_spec(rb, d2, 0), _part_spec(rb, d2, 1),
                  _rows_spec(rb, d2), _rows_spec(rb, 8),
                  _full_spec((1, d2)), _full_spec(fc2W.shape),
                  _full_spec((1, d2)), _full_spec((W3.shape[0], d3p))],
        out_specs=_rows_spec(rb, d3p),
        out_shape=jax.ShapeDtypeStruct((n, d3p), jnp.float32),
    )(a2p, a2p, u2, dinv, b2.reshape(1, -1), fc2W, fc2b.reshape(1, -1), W3p)

    a3p = _make_sc_agg(n, ep, d3p, tc_tiling=False)(u3, packed)

    b3p = jnp.zeros((1, d3p), jnp.float32).at[0, :nout].set(b3)
    fc3Wp = jnp.zeros((d3p, d3p), jnp.float32).at[:nout, :nout].set(fc3W)
    fc3bp = jnp.zeros((1, d3p), jnp.float32).at[0, :nout].set(fc3b)
    zp = pl.pallas_call(
        _out_body,
        grid=grid,
        in_specs=[_part_spec(rb, d3p, 0), _part_spec(rb, d3p, 1),
                  _rows_spec(rb, d3p), _rows_spec(rb, 8),
                  _full_spec((1, d3p)), _full_spec((d3p, d3p)),
                  _full_spec((1, d3p))],
        out_specs=_rows_spec(rb, d3p),
        out_shape=jax.ShapeDtypeStruct((n, d3p), jnp.float32),
    )(a3p, a3p, u3, dinv, b3p, fc3Wp, fc3bp)

    return zp[:, :nout]
